# baseline (speedup 1.0000x reference)
import jax
import jax.numpy as jnp
from jax.experimental import pallas as pl
from jax.experimental.pallas import tpu as pltpu

D = H = W = 14
HP = WP = 16
DP = 17
PLANE = HP * WP
NROWS = D * PLANE
NPAD = DP * PLANE
NVALID = D * H * W
OFF0 = PLANE + WP + 1
MT = 512


def _leaky(v):
    return jnp.where(v >= 0, v, 0.01 * v)


def _valid_mask():
    r = jax.lax.broadcasted_iota(jnp.int32, (NROWS, 1), 0)
    w = r & (WP - 1)
    h = (r // WP) & (HP - 1)
    return ((w < W) & (h < H)).astype(jnp.float32)


def _block_kernel(x64_ref, w1p_ref, wcen_ref, w2p_ref, w2s_ref, bsk_ref,
                  sse_ref, y_ref, s0_ref, s1_ref, s2_ref,
                  g0_ref, g1_ref, g2_ref, acc_ref, skip_ref):
    for s in range(4):
        _one_sample(s, x64_ref, w1p_ref, wcen_ref, w2p_ref, w2s_ref, bsk_ref,
                    sse_ref, y_ref, s0_ref, s1_ref, s2_ref,
                    g0_ref, g1_ref, g2_ref, acc_ref, skip_ref)


def _one_sample(s, x64_ref, w1p_ref, wcen_ref, w2p_ref, w2s_ref, bsk_ref,
                sse_ref, y_ref, s0_ref, s1_ref, s2_ref,
                g0_ref, g1_ref, g2_ref, acc_ref, skip_ref):
    mask = _valid_mask()

    s0_ref[:, 0:64] = x64_ref[s]
    s0_ref[:NPAD - 16, 64:128] = x64_ref[s, 16:, :]

    s1_ref[:NPAD - 16, :] = s0_ref[1:NPAD - 15, :]
    s2_ref[:NPAD - 16, :] = s0_ref[2:NPAD - 14, :]
    srcs = (s0_ref, s1_ref, s2_ref)
    sm = sq = None
    for m in range(0, NROWS, MT):
        acc = None
        for kd in range(3):
            for kw in range(3):
                idx = kd * 3 + kw
                base = kd * PLANE + m
                if kd == 1 and kw == 1:
                    part = jnp.dot(srcs[kw][base:base + MT, :], wcen_ref[...],
                                   preferred_element_type=jnp.float32)
                    skip_ref[m:m + MT, :] = part[:, 128:256]
                    part = part[:, 0:128]
                else:
                    part = jnp.dot(srcs[kw][base:base + MT, :],
                                   w1p_ref[idx * 128:(idx + 1) * 128, :],
                                   preferred_element_type=jnp.float32)
                acc = part if acc is None else acc + part
                solo = jnp.dot(srcs[kw][base + 2 * WP:base + 2 * WP + MT, :],
                               w1p_ref[(9 + idx) * 128:(10 + idx) * 128, :],
                               preferred_element_type=jnp.float32)
                acc = acc + solo
        acc = acc * mask[m:m + MT, :]
        acc_ref[m:m + MT, :] = acc
        psm = jnp.sum(acc, axis=0, keepdims=True)
        psq = jnp.sum(jnp.square(acc), axis=0, keepdims=True)
        sm = psm if sm is None else sm + psm
        sq = psq if sq is None else sq + psq

    mean1 = sm * (1.0 / NVALID)
    var1 = sq * (1.0 / NVALID) - jnp.square(mean1)
    r1 = jax.lax.rsqrt(var1 + 1e-5)
    o1b = (_leaky((acc_ref[...] - mean1) * r1) * mask).astype(jnp.bfloat16)
    zs = jnp.zeros((288, 256), jnp.bfloat16)
    zt = jnp.zeros((336, 256), jnp.bfloat16)
    for kw, g_ref in enumerate((g0_ref, g1_ref, g2_ref)):
        g_ref[0:288, :] = zs
        g_ref[3824:4160, :] = zt
        g_ref[OFF0 - kw:OFF0 - kw + NROWS, 0:128] = o1b
        g_ref[OFF0 - WP - kw:OFF0 - WP - kw + NROWS, 128:256] = o1b

    gs = (g0_ref, g1_ref, g2_ref)
    sm = sq = None
    for m in range(0, NROWS, MT):
        acc = None
        for kd in range(3):
            for kw in range(3):
                idx = kd * 3 + kw
                base = kd * PLANE + m
                part = jnp.dot(gs[kw][base:base + MT, :],
                               w2p_ref[idx * 256:(idx + 1) * 256, :],
                               preferred_element_type=jnp.float32)
                solo = jnp.dot(gs[kw][base + 2 * WP:base + 2 * WP + MT, 0:128],
                               w2s_ref[idx * 128:(idx + 1) * 128, :],
                               preferred_element_type=jnp.float32)
                part = part + solo
                acc = part if acc is None else acc + part
        acc = acc * mask[m:m + MT, :]
        acc_ref[m:m + MT, :] = acc
        psm = jnp.sum(acc, axis=0, keepdims=True)
        psq = jnp.sum(jnp.square(acc), axis=0, keepdims=True)
        sm = psm if sm is None else sm + psm
        sq = psq if sq is None else sq + psq

    mean2 = sm * (1.0 / NVALID)
    var2 = sq * (1.0 / NVALID) - jnp.square(mean2)
    scale = jax.lax.rsqrt(var2 + 1e-5) * sse_ref[...]
    y_ref[s] = _leaky((acc_ref[...] - mean2) * scale
                      + skip_ref[...] + bsk_ref[...]).astype(jnp.bfloat16)


def kernel(x, w_conv1, b_conv1, w_conv2, b_conv2, w_fc1, b_fc1,
           w_fc2, b_fc2, w_skip, b_skip):
    N, Cin = x.shape[0], x.shape[1]
    Cout = w_conv1.shape[0]

    xt = jnp.transpose(x, (0, 2, 3, 4, 1))
    xf = jnp.pad(xt, ((0, 0), (1, DP - 1 - D), (1, HP - 1 - H),
                      (1, WP - 1 - W), (0, 0)))
    xp = xf.reshape(N, NPAD, Cin).astype(jnp.bfloat16)

    w1 = jnp.transpose(w_conv1, (2, 3, 4, 1, 0)).astype(jnp.bfloat16)
    w2 = jnp.transpose(w_conv2, (2, 3, 4, 1, 0)).astype(jnp.bfloat16)
    zk = jnp.zeros((Cin, Cout), jnp.bfloat16)
    w1p = jnp.concatenate(
        [jnp.concatenate([w1[kd, 0, kw], w1[kd, 1, kw]], axis=0)
         for kd in range(3) for kw in range(3)] +
        [jnp.concatenate([w1[kd, 2, kw], zk], axis=0)
         for kd in range(3) for kw in range(3)], axis=0)
    wsk = w_skip.reshape(Cout, Cin).T.astype(jnp.bfloat16)
    wcen = jnp.concatenate(
        [w1p[4 * 128:5 * 128, :],
         jnp.concatenate([jnp.zeros((Cin, Cout), jnp.bfloat16), wsk], axis=0)],
        axis=1)
    w2p = jnp.concatenate(
        [jnp.concatenate([w2[kd, 0, kw], w2[kd, 1, kw]], axis=0)
         for kd in range(3) for kw in range(3)], axis=0)
    w2s = jnp.concatenate(
        [w2[kd, 2, kw] for kd in range(3) for kw in range(3)],
        axis=0)

    bsk = b_skip.reshape(1, Cout)
    sse = jax.nn.sigmoid(
        jnp.maximum(b_fc1, 0.0)[None, :] @ w_fc2.T + b_fc2[None, :])

    y = pl.pallas_call(
        _block_kernel,
        out_shape=jax.ShapeDtypeStruct((N, NROWS, 128), jnp.bfloat16),
        grid=(N // 4,),
        in_specs=[
            pl.BlockSpec((4, NPAD, 64), lambda n: (n, 0, 0)),
            pl.BlockSpec((18 * 128, 128), lambda n: (0, 0)),
            pl.BlockSpec((128, 256), lambda n: (0, 0)),
            pl.BlockSpec((9 * 256, 128), lambda n: (0, 0)),
            pl.BlockSpec((9 * 128, 128), lambda n: (0, 0)),
            pl.BlockSpec((1, 128), lambda n: (0, 0)),
            pl.BlockSpec((1, 128), lambda n: (0, 0)),
        ],
        out_specs=pl.BlockSpec((4, NROWS, 128), lambda n: (n, 0, 0)),
        scratch_shapes=[
            pltpu.VMEM((NPAD, 128), jnp.bfloat16),
            pltpu.VMEM((NPAD, 128), jnp.bfloat16),
            pltpu.VMEM((NPAD, 128), jnp.bfloat16),
            pltpu.VMEM((NPAD, 256), jnp.bfloat16),
            pltpu.VMEM((NPAD, 256), jnp.bfloat16),
            pltpu.VMEM((NPAD, 256), jnp.bfloat16),
            pltpu.VMEM((NROWS, 128), jnp.float32),
            pltpu.VMEM((NROWS, 128), jnp.float32),
        ],
        compiler_params=pltpu.CompilerParams(
            dimension_semantics=("parallel",),
            vmem_limit_bytes=96 * 1024 * 1024),
    )(xp, w1p, wcen, w2p, w2s, bsk, sse)

    y = y.reshape(N, D, HP, WP, 128)[:, :, :H, :W, :Cout]
    return jnp.transpose(y, (0, 4, 1, 2, 3)).astype(jnp.float32)


# speedup vs baseline: 1.1537x; 1.1537x over previous
import jax
import jax.numpy as jnp
from jax.experimental import pallas as pl
from jax.experimental.pallas import tpu as pltpu

D = H = W = 14
HP = WP = 16
DP = 17
PLANE = HP * WP
NROWS = D * PLANE
NPAD = DP * PLANE
NVALID = D * H * W
OFF0 = PLANE + WP + 1
MT = 512


def _leaky(v):
    return jnp.where(v >= 0, v, 0.01 * v)


def _valid_mask():
    r = jax.lax.broadcasted_iota(jnp.int32, (NROWS, 1), 0)
    w = r & (WP - 1)
    h = (r // WP) & (HP - 1)
    return ((w < W) & (h < H)).astype(jnp.float32)


def _block_kernel(x64_ref, w1p_ref, wcen_ref, w2p_ref, w2s_ref, bsk_ref,
                  sse_ref, y_ref, s0_ref, s1_ref, s2_ref,
                  g0_ref, g1_ref, g2_ref, acc_ref, skip_ref):
    for s in range(2):
        _one_sample(s, x64_ref, w1p_ref, wcen_ref, w2p_ref, w2s_ref, bsk_ref,
                    sse_ref, y_ref, s0_ref, s1_ref, s2_ref,
                    g0_ref, g1_ref, g2_ref, acc_ref, skip_ref)


def _one_sample(s, x64_ref, w1p_ref, wcen_ref, w2p_ref, w2s_ref, bsk_ref,
                sse_ref, y_ref, s0_ref, s1_ref, s2_ref,
                g0_ref, g1_ref, g2_ref, acc_ref, skip_ref):
    mask = _valid_mask()

    s0_ref[:, 0:64] = x64_ref[s]
    s0_ref[:NPAD - 16, 64:128] = x64_ref[s, 16:, :]

    s1_ref[:NPAD - 16, :] = s0_ref[1:NPAD - 15, :]
    s2_ref[:NPAD - 16, :] = s0_ref[2:NPAD - 14, :]
    srcs = (s0_ref, s1_ref, s2_ref)
    sm = sq = None
    for m in range(0, NROWS, MT):
        acc = None
        for kd in range(3):
            for kw in range(3):
                idx = kd * 3 + kw
                base = kd * PLANE + m
                if kd == 1 and kw == 1:
                    part = jnp.dot(srcs[kw][base:base + MT, :], wcen_ref[...],
                                   preferred_element_type=jnp.float32)
                    skip_ref[m:m + MT, :] = part[:, 128:256].astype(jnp.bfloat16)
                    part = part[:, 0:128]
                else:
                    part = jnp.dot(srcs[kw][base:base + MT, :],
                                   w1p_ref[idx * 128:(idx + 1) * 128, :],
                                   preferred_element_type=jnp.float32)
                acc = part if acc is None else acc + part
                solo = jnp.dot(srcs[kw][base + 2 * WP:base + 2 * WP + MT, :],
                               w1p_ref[(9 + idx) * 128:(10 + idx) * 128, :],
                               preferred_element_type=jnp.float32)
                acc = acc + solo
        acc = acc * mask[m:m + MT, :]
        acc_ref[m:m + MT, :] = acc
        psm = jnp.sum(acc, axis=0, keepdims=True)
        psq = jnp.sum(jnp.square(acc), axis=0, keepdims=True)
        sm = psm if sm is None else sm + psm
        sq = psq if sq is None else sq + psq

    mean1 = sm * (1.0 / NVALID)
    var1 = sq * (1.0 / NVALID) - jnp.square(mean1)
    r1 = jax.lax.rsqrt(var1 + 1e-5)
    o1b = (_leaky((acc_ref[...] - mean1) * r1) * mask).astype(jnp.bfloat16)
    zs = jnp.zeros((288, 256), jnp.bfloat16)
    zt = jnp.zeros((336, 256), jnp.bfloat16)
    for kw, g_ref in enumerate((g0_ref, g1_ref, g2_ref)):
        g_ref[0:288, :] = zs
        g_ref[3824:4160, :] = zt
        g_ref[OFF0 - kw:OFF0 - kw + NROWS, 0:128] = o1b
        g_ref[OFF0 - WP - kw:OFF0 - WP - kw + NROWS, 128:256] = o1b

    gs = (g0_ref, g1_ref, g2_ref)
    sm = sq = None
    for m in range(0, NROWS, MT):
        acc = None
        for kd in range(3):
            for kw in range(3):
                idx = kd * 3 + kw
                base = kd * PLANE + m
                part = jnp.dot(gs[kw][base:base + MT, :],
                               w2p_ref[idx * 256:(idx + 1) * 256, :],
                               preferred_element_type=jnp.float32)
                solo = jnp.dot(gs[kw][base + 2 * WP:base + 2 * WP + MT, 0:128],
                               w2s_ref[idx * 128:(idx + 1) * 128, :],
                               preferred_element_type=jnp.float32)
                part = part + solo
                acc = part if acc is None else acc + part
        acc = acc * mask[m:m + MT, :]
        acc_ref[m:m + MT, :] = acc
        psm = jnp.sum(acc, axis=0, keepdims=True)
        psq = jnp.sum(jnp.square(acc), axis=0, keepdims=True)
        sm = psm if sm is None else sm + psm
        sq = psq if sq is None else sq + psq

    mean2 = sm * (1.0 / NVALID)
    var2 = sq * (1.0 / NVALID) - jnp.square(mean2)
    scale = jax.lax.rsqrt(var2 + 1e-5) * sse_ref[...]
    y_ref[s] = _leaky((acc_ref[...] - mean2) * scale
                      + skip_ref[...] + bsk_ref[...]).astype(jnp.bfloat16)


def kernel(x, w_conv1, b_conv1, w_conv2, b_conv2, w_fc1, b_fc1,
           w_fc2, b_fc2, w_skip, b_skip):
    N, Cin = x.shape[0], x.shape[1]
    Cout = w_conv1.shape[0]

    xt = jnp.transpose(x, (0, 2, 3, 4, 1))
    xf = jnp.pad(xt, ((0, 0), (1, DP - 1 - D), (1, HP - 1 - H),
                      (1, WP - 1 - W), (0, 0)))
    xp = xf.reshape(N, NPAD, Cin).astype(jnp.bfloat16)

    w1 = jnp.transpose(w_conv1, (2, 3, 4, 1, 0)).astype(jnp.bfloat16)
    w2 = jnp.transpose(w_conv2, (2, 3, 4, 1, 0)).astype(jnp.bfloat16)
    zk = jnp.zeros((Cin, Cout), jnp.bfloat16)
    w1p = jnp.concatenate(
        [jnp.concatenate([w1[kd, 0, kw], w1[kd, 1, kw]], axis=0)
         for kd in range(3) for kw in range(3)] +
        [jnp.concatenate([w1[kd, 2, kw], zk], axis=0)
         for kd in range(3) for kw in range(3)], axis=0)
    wsk = w_skip.reshape(Cout, Cin).T.astype(jnp.bfloat16)
    wcen = jnp.concatenate(
        [w1p[4 * 128:5 * 128, :],
         jnp.concatenate([jnp.zeros((Cin, Cout), jnp.bfloat16), wsk], axis=0)],
        axis=1)
    w2p = jnp.concatenate(
        [jnp.concatenate([w2[kd, 0, kw], w2[kd, 1, kw]], axis=0)
         for kd in range(3) for kw in range(3)], axis=0)
    w2s = jnp.concatenate(
        [w2[kd, 2, kw] for kd in range(3) for kw in range(3)],
        axis=0)

    bsk = b_skip.reshape(1, Cout)
    sse = jax.nn.sigmoid(
        jnp.maximum(b_fc1, 0.0)[None, :] @ w_fc2.T + b_fc2[None, :])

    y = pl.pallas_call(
        _block_kernel,
        out_shape=jax.ShapeDtypeStruct((N, NROWS, 128), jnp.bfloat16),
        grid=(N // 2,),
        in_specs=[
            pl.BlockSpec((2, NPAD, 64), lambda n: (n, 0, 0)),
            pl.BlockSpec((18 * 128, 128), lambda n: (0, 0)),
            pl.BlockSpec((128, 256), lambda n: (0, 0)),
            pl.BlockSpec((9 * 256, 128), lambda n: (0, 0)),
            pl.BlockSpec((9 * 128, 128), lambda n: (0, 0)),
            pl.BlockSpec((1, 128), lambda n: (0, 0)),
            pl.BlockSpec((1, 128), lambda n: (0, 0)),
        ],
        out_specs=pl.BlockSpec((2, NROWS, 128), lambda n: (n, 0, 0)),
        scratch_shapes=[
            pltpu.VMEM((NPAD, 128), jnp.bfloat16),
            pltpu.VMEM((NPAD, 128), jnp.bfloat16),
            pltpu.VMEM((NPAD, 128), jnp.bfloat16),
            pltpu.VMEM((NPAD, 256), jnp.bfloat16),
            pltpu.VMEM((NPAD, 256), jnp.bfloat16),
            pltpu.VMEM((NPAD, 256), jnp.bfloat16),
            pltpu.VMEM((NROWS, 128), jnp.float32),
            pltpu.VMEM((NROWS, 128), jnp.bfloat16),
        ],
        compiler_params=pltpu.CompilerParams(
            dimension_semantics=("parallel",),
            vmem_limit_bytes=96 * 1024 * 1024),
    )(xp, w1p, wcen, w2p, w2s, bsk, sse)

    y = y.reshape(N, D, HP, WP, 128)[:, :, :H, :W, :Cout]
    return jnp.transpose(y, (0, 4, 1, 2, 3)).astype(jnp.float32)
